# Initial kernel scaffold; baseline (speedup 1.0000x reference)
import jax
import jax.numpy as jnp
from jax.experimental import pallas as pl


def kernel(x, edge_index, W1, a1_src, a1_dst, b1, W_ed, W2, a2_src, a2_dst, b2):
    raise NotImplementedError("write your pallas kernel here")


# speedup vs baseline: 53.3196x; 53.3196x over previous
import functools

import jax
import jax.numpy as jnp
from jax import lax
from jax.experimental import pallas as pl
from jax.experimental.pallas import tpu as pltpu
from jax.experimental.pallas import tpu_sc as plsc

N = 10000
E = 320000
D_IN = 128
D_HID = 8
NC = 2
NS = 16
NW = NC * NS
EPW = E // NW
B = 2000
CH = EPW // B
W_ACC = D_HID + 1

_f32 = jnp.float32
_i32 = jnp.int32


def _leaky(v, slope):
    return jnp.maximum(v, slope * v)


def _stage1_body(x_ref, w1_ref, as_ref, ad_ref, pk_ref, nt_ref):
    h = jnp.dot(x_ref[...], w1_ref[...], preferred_element_type=_f32)
    s = jnp.dot(h, as_ref[...], preferred_element_type=_f32)
    d = jnp.dot(h, ad_ref[...], preferred_element_type=_f32)
    pk_ref[...] = jnp.concatenate(
        [s, h, jnp.zeros((N, 7), _f32)], axis=1)
    nt_ref[...] = jnp.concatenate([s, d, h], axis=1).T


def _stage1(x, w1, a_s, a_d):
    return pl.pallas_call(
        _stage1_body,
        out_shape=[
            jax.ShapeDtypeStruct((N, 16), _f32),
            jax.ShapeDtypeStruct((10, N), _f32),
        ],
    )(x, w1, a_s.reshape(D_HID, 1), a_d.reshape(D_HID, 1))


_mesh = plsc.VectorSubcoreMesh(
    core_axis_name="c", subcore_axis_name="s", num_cores=NC, num_subcores=NS)


@functools.partial(
    pl.kernel,
    out_type=jax.ShapeDtypeStruct((NW, W_ACC * N), _f32),
    mesh=_mesh,
    compiler_params=pltpu.CompilerParams(
        needs_layout_passes=False, use_tc_tiling_on_sc=False),
    scratch_types=[
        pltpu.VMEM((W_ACC * N,), _f32),
        pltpu.VMEM((B,), _i32),
        pltpu.VMEM((B,), _i32),
        pltpu.VMEM((B, 16), _f32),
        pltpu.VMEM((B,), _f32),
        pltpu.SemaphoreType.DMA,
        pltpu.SemaphoreType.DMA,
    ],
)
def _edge_pass(pk_hbm, d_hbm, src_hbm, dst_hbm, out_hbm,
               acc_v, sidx_v, didx_v, rows_v, dval_v, sem1, sem2):
    cid = lax.axis_index("c")
    sid = lax.axis_index("s")
    w = sid * NC + cid

    def zero_body(i, carry):
        acc_v[pl.ds(i * 16, 16)] = jnp.zeros((16,), _f32)
        return carry

    lax.fori_loop(0, (W_ACC * N) // 16, zero_body, 0)

    lane = jnp.arange(16, dtype=_i32)
    ebase = w * EPW
    for ci in range(CH):
        base = pl.multiple_of(ebase + ci * B, B)
        pltpu.sync_copy(src_hbm.at[pl.ds(base, B)], sidx_v)
        pltpu.sync_copy(dst_hbm.at[pl.ds(base, B)], didx_v)
        cp1 = pltpu.async_copy(pk_hbm.at[sidx_v], rows_v, sem1)
        cp2 = pltpu.async_copy(d_hbm.at[didx_v], dval_v, sem2)
        cp1.wait()
        cp2.wait()

        def group_body(g, carry):
            r0 = g * 16
            row_ids = r0 + lane
            s = plsc.load_gather(rows_v, [row_ids, jnp.zeros((16,), _i32)])
            d = dval_v[pl.ds(r0, 16)]
            t = s + d
            ex = jnp.exp(jnp.maximum(t, 0.2 * t))
            di = didx_v[pl.ds(r0, 16)]
            plsc.addupdate_scatter(acc_v, [di + (D_HID * N)], ex)
            for c in range(D_HID):
                hc = plsc.load_gather(
                    rows_v, [row_ids, jnp.full((16,), c + 1, _i32)])
                plsc.addupdate_scatter(acc_v, [di + (c * N)], ex * hc)
            return carry

        lax.fori_loop(0, B // 16, group_body, 0)

    pltpu.sync_copy(acc_v, out_hbm.at[w])


def _stage3_body(part_ref, nt1_ref, b1_ref, wed_ref, w2_ref, a2s_ref,
                 a2d_ref, pk2_ref, nt2_ref, acc_ref):
    i = pl.program_id(0)

    @pl.when(i == 0)
    def _():
        acc_ref[...] = jnp.zeros((W_ACC, N), _f32)

    acc_ref[...] += part_ref[0]

    @pl.when(i == NW - 1)
    def _():
        red = acc_ref[...]
        s1 = nt1_ref[0:1, :]
        d1 = nt1_ref[1:2, :]
        h1 = nt1_ref[2:10, :]
        t = s1 + d1
        ex = jnp.exp(jnp.maximum(t, 0.2 * t))
        num = red[0:D_HID] + ex * h1
        den = red[D_HID:W_ACC] + ex
        enc = num / (den + 1e-16) + b1_ref[...]
        enc = _leaky(enc, 0.01)
        rep = lax.dot_general(wed_ref[...], enc, (((0,), (0,)), ((), ())),
                              preferred_element_type=_f32)
        c_s = jnp.dot(w2_ref[...], a2s_ref[...],
                      preferred_element_type=_f32)
        c_d = jnp.dot(w2_ref[...], a2d_ref[...],
                      preferred_element_type=_f32)
        s2 = jnp.sum(rep * c_s, axis=0, keepdims=True)
        d2 = jnp.sum(rep * c_d, axis=0, keepdims=True)
        pk2_ref[...] = jnp.concatenate(
            [s2, rep, jnp.zeros((7, N), _f32)], axis=0).T
        nt2_ref[...] = jnp.concatenate([s2, d2, rep], axis=0)


def _stage3(part, nt1, b1, wed, w2, a2s, a2d):
    return pl.pallas_call(
        _stage3_body,
        grid=(NW,),
        in_specs=[
            pl.BlockSpec((1, W_ACC, N), lambda i: (i, 0, 0)),
            pl.BlockSpec((10, N), lambda i: (0, 0)),
            pl.BlockSpec((D_HID, 1), lambda i: (0, 0)),
            pl.BlockSpec((D_HID, D_HID), lambda i: (0, 0)),
            pl.BlockSpec((D_HID, D_IN), lambda i: (0, 0)),
            pl.BlockSpec((D_IN, 1), lambda i: (0, 0)),
            pl.BlockSpec((D_IN, 1), lambda i: (0, 0)),
        ],
        out_specs=[
            pl.BlockSpec((N, 16), lambda i: (0, 0)),
            pl.BlockSpec((10, N), lambda i: (0, 0)),
        ],
        out_shape=[
            jax.ShapeDtypeStruct((N, 16), _f32),
            jax.ShapeDtypeStruct((10, N), _f32),
        ],
        scratch_shapes=[pltpu.VMEM((W_ACC, N), _f32)],
    )(part, nt1, b1.reshape(D_HID, 1), wed, w2,
      a2s.reshape(D_IN, 1), a2d.reshape(D_IN, 1))


def _stage5_body(part_ref, nt2_ref, w2_ref, b2_ref, out_ref, acc_ref):
    i = pl.program_id(0)

    @pl.when(i == 0)
    def _():
        acc_ref[...] = jnp.zeros((W_ACC, N), _f32)

    acc_ref[...] += part_ref[0]

    @pl.when(i == NW - 1)
    def _():
        red = acc_ref[...]
        s2 = nt2_ref[0:1, :]
        d2 = nt2_ref[1:2, :]
        rep = nt2_ref[2:10, :]
        t = s2 + d2
        ex = jnp.exp(jnp.maximum(t, 0.2 * t))
        agg_t = (red[0:D_HID] + ex * rep) / (red[D_HID:W_ACC] + ex + 1e-16)
        agg = agg_t.T
        y = jnp.dot(agg, w2_ref[...], preferred_element_type=_f32)
        y = y + b2_ref[...]
        out_ref[...] = _leaky(y, 0.01)


def _stage5(part, nt2, w2, b2):
    return pl.pallas_call(
        _stage5_body,
        grid=(NW,),
        in_specs=[
            pl.BlockSpec((1, W_ACC, N), lambda i: (i, 0, 0)),
            pl.BlockSpec((10, N), lambda i: (0, 0)),
            pl.BlockSpec((D_HID, D_IN), lambda i: (0, 0)),
            pl.BlockSpec((1, D_IN), lambda i: (0, 0)),
        ],
        out_specs=pl.BlockSpec((N, D_IN), lambda i: (0, 0)),
        out_shape=jax.ShapeDtypeStruct((N, D_IN), _f32),
        scratch_shapes=[pltpu.VMEM((W_ACC, N), _f32)],
    )(part, nt2, w2, b2.reshape(1, D_IN))


def kernel(x, edge_index, W1, a1_src, a1_dst, b1, W_ed, W2, a2_src, a2_dst,
           b2):
    src = edge_index[0].astype(_i32)
    dst = edge_index[1].astype(_i32)
    pk1, nt1 = _stage1(x, W1, a1_src, a1_dst)
    d1 = nt1[1]
    part1 = _edge_pass(pk1, d1, src, dst).reshape(NW, W_ACC, N)
    pk2, nt2 = _stage3(part1, nt1, b1, W_ed, W2, a2_src, a2_dst)
    d2 = nt2[1]
    part2 = _edge_pass(pk2, d2, src, dst).reshape(NW, W_ACC, N)
    return _stage5(part2, nt2, W2, b2)


# speedup vs baseline: 96.3243x; 1.8065x over previous
import functools

import jax
import jax.numpy as jnp
from jax import lax
from jax.experimental import pallas as pl
from jax.experimental.pallas import tpu as pltpu
from jax.experimental.pallas import tpu_sc as plsc

N = 10000
E = 320000
D_IN = 128
D_HID = 8
NC = 2
NS = 16
NW = NC * NS
EPW = E // NW
UB = 400
UNITS = EPW // UB
UG = UB // 16
W_ACC = D_HID + 1
NP = 8
_f32 = jnp.float32
_i32 = jnp.int32


def _leaky(v, slope):
    return jnp.maximum(v, slope * v)


def _stage1_body(x_ref, w1_ref, as_ref, ad_ref, h_ref, nt_ref):
    h = jnp.dot(x_ref[...], w1_ref[...], preferred_element_type=_f32)
    s = jnp.dot(h, as_ref[...], preferred_element_type=_f32)
    d = jnp.dot(h, ad_ref[...], preferred_element_type=_f32)
    h_ref[...] = h
    nt_ref[...] = jnp.concatenate([s, d, h], axis=1).T


def _stage1(x, w1, a_s, a_d):
    return pl.pallas_call(
        _stage1_body,
        out_shape=[
            jax.ShapeDtypeStruct((N, D_HID), _f32),
            jax.ShapeDtypeStruct((10, N), _f32),
        ],
    )(x, w1, a_s.reshape(D_HID, 1), a_d.reshape(D_HID, 1))


_mesh = plsc.VectorSubcoreMesh(
    core_axis_name="c", subcore_axis_name="s", num_cores=NC, num_subcores=NS)


@functools.partial(
    pl.kernel,
    out_type=jax.ShapeDtypeStruct((NW, W_ACC * N), _f32),
    mesh=_mesh,
    compiler_params=pltpu.CompilerParams(
        needs_layout_passes=False, use_tc_tiling_on_sc=False),
    scratch_types=[
        pltpu.VMEM((W_ACC * N,), _f32),
        pltpu.VMEM((N,), _f32),
        pltpu.VMEM((N,), _f32),
        pltpu.VMEM((3, UB), _i32),
        pltpu.VMEM((3, UB), _i32),
        pltpu.VMEM((2, UB, D_HID), _f32),
        pltpu.SemaphoreType.DMA,
        pltpu.SemaphoreType.DMA,
        pltpu.SemaphoreType.DMA,
        pltpu.SemaphoreType.DMA,
        pltpu.SemaphoreType.DMA,
    ],
)
def _edge_pass(h_hbm, s_hbm, d_hbm, src_hbm, dst_hbm, out_hbm,
               acc_v, s_loc, d_loc, sidx_v, didx_v, rows_v,
               semi0, semi1, semi2, semr0, semr1):
    cid = lax.axis_index("c")
    sid = lax.axis_index("s")
    w = sid * NC + cid
    semi = [semi0, semi1, semi2]
    semr = [semr0, semr1]

    pltpu.sync_copy(s_hbm, s_loc)
    pltpu.sync_copy(d_hbm, d_loc)

    @plsc.parallel_loop(0, (W_ACC * N) // 16, unroll=8)
    def zero_body(i):
        acc_v[pl.ds(i * 16, 16)] = jnp.zeros((16,), _f32)

    lane = jnp.arange(16, dtype=_i32)
    ebase = w * EPW
    d_is = [None] * UNITS
    d_id = [None] * UNITS
    d_r = [None] * UNITS

    def fire_idx(u):
        j = u % 3
        base = pl.multiple_of(ebase + u * UB, UB)
        d_is[u] = pltpu.async_copy(
            src_hbm.at[pl.ds(base, UB)], sidx_v.at[j], semi[j])
        d_id[u] = pltpu.async_copy(
            dst_hbm.at[pl.ds(base, UB)], didx_v.at[j], semi[j])

    def fire_rows(u):
        d_is[u].wait()
        d_id[u].wait()
        d_r[u] = pltpu.async_copy(
            h_hbm.at[sidx_v.at[u % 3]], rows_v.at[u % 2], semr[u % 2])

    fire_idx(0)
    fire_idx(1)
    fire_rows(0)
    for u in range(UNITS):
        if u + 2 < UNITS:
            fire_idx(u + 2)
        d_r[u].wait()
        if u + 1 < UNITS:
            fire_rows(u + 1)
        sidx_u = sidx_v.at[u % 3]
        didx_u = didx_v.at[u % 3]
        rows_u = rows_v.at[u % 2]

        @plsc.parallel_loop(0, UG, unroll=1)
        def group_body(g):
            r0 = g * 16
            row_ids = r0 + lane
            si = sidx_u[pl.ds(r0, 16)]
            di = didx_u[pl.ds(r0, 16)]
            s = plsc.load_gather(s_loc, [si])
            d = plsc.load_gather(d_loc, [di])
            t = s + d
            ex = jnp.exp(jnp.maximum(t, 0.2 * t))
            plsc.addupdate_scatter(acc_v, [di + (D_HID * N)], ex)
            for c in range(D_HID):
                hc = plsc.load_gather(
                    rows_u, [row_ids, jnp.full((16,), c, _i32)])
                plsc.addupdate_scatter(acc_v, [di + (c * N)], ex * hc)

    pltpu.sync_copy(acc_v, out_hbm.at[w])


def _stage3_body(part_ref, nt1_ref, b1_ref, wed_ref, w2_ref, a2s_ref,
                 a2d_ref, rep_ref, nt2_ref, acc_ref):
    i = pl.program_id(0)

    @pl.when(i == 0)
    def _():
        acc_ref[...] = jnp.zeros((W_ACC, N), _f32)

    acc_ref[...] += jnp.sum(part_ref[...], axis=0)

    @pl.when(i == (NW // NP) - 1)
    def _():
        red = acc_ref[...]
        s1 = nt1_ref[0:1, :]
        d1 = nt1_ref[1:2, :]
        h1 = nt1_ref[2:10, :]
        t = s1 + d1
        ex = jnp.exp(jnp.maximum(t, 0.2 * t))
        num = red[0:D_HID] + ex * h1
        den = red[D_HID:W_ACC] + ex
        enc = num / (den + 1e-16) + b1_ref[...]
        enc = _leaky(enc, 0.01)
        rep = lax.dot_general(wed_ref[...], enc, (((0,), (0,)), ((), ())),
                              preferred_element_type=_f32)
        c_s = jnp.dot(w2_ref[...], a2s_ref[...],
                      preferred_element_type=_f32)
        c_d = jnp.dot(w2_ref[...], a2d_ref[...],
                      preferred_element_type=_f32)
        s2 = jnp.sum(rep * c_s, axis=0, keepdims=True)
        d2 = jnp.sum(rep * c_d, axis=0, keepdims=True)
        rep_ref[...] = rep.T
        nt2_ref[...] = jnp.concatenate([s2, d2, rep], axis=0)


def _stage3(part, nt1, b1, wed, w2, a2s, a2d):
    return pl.pallas_call(
        _stage3_body,
        grid=(NW // NP,),
        in_specs=[
            pl.BlockSpec((NP, W_ACC, N), lambda i: (i, 0, 0)),
            pl.BlockSpec((10, N), lambda i: (0, 0)),
            pl.BlockSpec((D_HID, 1), lambda i: (0, 0)),
            pl.BlockSpec((D_HID, D_HID), lambda i: (0, 0)),
            pl.BlockSpec((D_HID, D_IN), lambda i: (0, 0)),
            pl.BlockSpec((D_IN, 1), lambda i: (0, 0)),
            pl.BlockSpec((D_IN, 1), lambda i: (0, 0)),
        ],
        out_specs=[
            pl.BlockSpec((N, D_HID), lambda i: (0, 0)),
            pl.BlockSpec((10, N), lambda i: (0, 0)),
        ],
        out_shape=[
            jax.ShapeDtypeStruct((N, D_HID), _f32),
            jax.ShapeDtypeStruct((10, N), _f32),
        ],
        scratch_shapes=[pltpu.VMEM((W_ACC, N), _f32)],
    )(part, nt1, b1.reshape(D_HID, 1), wed, w2,
      a2s.reshape(D_IN, 1), a2d.reshape(D_IN, 1))


def _stage5_body(part_ref, nt2_ref, w2_ref, b2_ref, out_ref, acc_ref):
    i = pl.program_id(0)

    @pl.when(i == 0)
    def _():
        acc_ref[...] = jnp.zeros((W_ACC, N), _f32)

    acc_ref[...] += jnp.sum(part_ref[...], axis=0)

    @pl.when(i == (NW // NP) - 1)
    def _():
        red = acc_ref[...]
        s2 = nt2_ref[0:1, :]
        d2 = nt2_ref[1:2, :]
        rep = nt2_ref[2:10, :]
        t = s2 + d2
        ex = jnp.exp(jnp.maximum(t, 0.2 * t))
        agg_t = (red[0:D_HID] + ex * rep) / (red[D_HID:W_ACC] + ex + 1e-16)
        agg = agg_t.T
        y = jnp.dot(agg, w2_ref[...], preferred_element_type=_f32)
        y = y + b2_ref[...]
        out_ref[...] = _leaky(y, 0.01)


def _stage5(part, nt2, w2, b2):
    return pl.pallas_call(
        _stage5_body,
        grid=(NW // NP,),
        in_specs=[
            pl.BlockSpec((NP, W_ACC, N), lambda i: (i, 0, 0)),
            pl.BlockSpec((10, N), lambda i: (0, 0)),
            pl.BlockSpec((D_HID, D_IN), lambda i: (0, 0)),
            pl.BlockSpec((1, D_IN), lambda i: (0, 0)),
        ],
        out_specs=pl.BlockSpec((N, D_IN), lambda i: (0, 0)),
        out_shape=jax.ShapeDtypeStruct((N, D_IN), _f32),
        scratch_shapes=[pltpu.VMEM((W_ACC, N), _f32)],
    )(part, nt2, w2, b2.reshape(1, D_IN))


def kernel(x, edge_index, W1, a1_src, a1_dst, b1, W_ed, W2, a2_src, a2_dst,
           b2):
    src = edge_index[0].astype(_i32)
    dst = edge_index[1].astype(_i32)
    h1, nt1 = _stage1(x, W1, a1_src, a1_dst)
    part1 = _edge_pass(h1, nt1[0], nt1[1], src, dst).reshape(NW, W_ACC, N)
    rep, nt2 = _stage3(part1, nt1, b1, W_ed, W2, a2_src, a2_dst)
    part2 = _edge_pass(rep, nt2[0], nt2[1], src, dst).reshape(NW, W_ACC, N)
    return _stage5(part2, nt2, W2, b2)


# speedup vs baseline: 118.7388x; 1.2327x over previous
import functools

import jax
import jax.numpy as jnp
from jax import lax
from jax.experimental import pallas as pl
from jax.experimental.pallas import tpu as pltpu
from jax.experimental.pallas import tpu_sc as plsc

N = 10000
E = 320000
D_IN = 128
D_HID = 8
NC = 2
NS = 16
NW = NC * NS
EPW = E // NW
UB = 400
UNITS = EPW // UB
UG = UB // 16
W_ACC = D_HID + 1
NP = 8
_f32 = jnp.float32
_i32 = jnp.int32


def _leaky(v, slope):
    return jnp.maximum(v, slope * v)


def _stage1_body(x_ref, w1_ref, as_ref, ad_ref, h_ref, nt_ref):
    h = jnp.dot(x_ref[...], w1_ref[...], preferred_element_type=_f32)
    s = jnp.dot(h, as_ref[...], preferred_element_type=_f32)
    d = jnp.dot(h, ad_ref[...], preferred_element_type=_f32)
    h_ref[...] = h
    nt_ref[...] = jnp.concatenate([s, d, h], axis=1).T


def _stage1(x, w1, a_s, a_d):
    return pl.pallas_call(
        _stage1_body,
        out_shape=[
            jax.ShapeDtypeStruct((N, D_HID), _f32),
            jax.ShapeDtypeStruct((10, N), _f32),
        ],
    )(x, w1, a_s.reshape(D_HID, 1), a_d.reshape(D_HID, 1))


_mesh = plsc.VectorSubcoreMesh(
    core_axis_name="c", subcore_axis_name="s", num_cores=NC, num_subcores=NS)


@functools.partial(
    pl.kernel,
    out_type=jax.ShapeDtypeStruct((NW, W_ACC, N), _f32),
    mesh=_mesh,
    compiler_params=pltpu.CompilerParams(
        needs_layout_passes=False, use_tc_tiling_on_sc=False),
    scratch_types=[
        pltpu.VMEM((W_ACC, N), _f32),
        pltpu.VMEM((N,), _f32),
        pltpu.VMEM((N,), _f32),
        pltpu.VMEM((4, UB), _i32),
        pltpu.VMEM((4, UB), _i32),
        pltpu.VMEM((3, UB, D_HID), _f32),
        pltpu.SemaphoreType.DMA,
        pltpu.SemaphoreType.DMA,
        pltpu.SemaphoreType.DMA,
        pltpu.SemaphoreType.DMA,
        pltpu.SemaphoreType.DMA,
        pltpu.SemaphoreType.DMA,
        pltpu.SemaphoreType.DMA,
        pltpu.SemaphoreType.DMA,
    ],
)
def _edge_pass(h_hbm, nt_hbm, edge_hbm, out_hbm,
               acc_v, s_loc, d_loc, sidx_v, didx_v, rows_v,
               semi0, semi1, semi2, semi3, semr0, semr1, semr2, semt):
    cid = lax.axis_index("c")
    sid = lax.axis_index("s")
    w = sid * NC + cid
    semi = [semi0, semi1, semi2, semi3]
    semr = [semr0, semr1, semr2]

    d_s = pltpu.async_copy(nt_hbm.at[0], s_loc, semt)
    d_d = pltpu.async_copy(nt_hbm.at[1], d_loc, semt)

    @plsc.parallel_loop(0, N // 16, unroll=4)
    def zero_body(i):
        for c in range(W_ACC):
            acc_v[c, pl.ds(i * 16, 16)] = jnp.zeros((16,), _f32)

    d_s.wait()
    d_d.wait()

    lane = jnp.arange(16, dtype=_i32)
    ebase = w * EPW
    d_is = [None] * UNITS
    d_id = [None] * UNITS
    d_r = [None] * UNITS

    def fire_idx(u):
        j = u % 4
        base = pl.multiple_of(ebase + u * UB, UB)
        d_is[u] = pltpu.async_copy(
            edge_hbm.at[0, pl.ds(base, UB)], sidx_v.at[j], semi[j])
        d_id[u] = pltpu.async_copy(
            edge_hbm.at[1, pl.ds(base, UB)], didx_v.at[j], semi[j])

    def fire_rows(u):
        d_is[u].wait()
        d_id[u].wait()
        d_r[u] = pltpu.async_copy(
            h_hbm.at[sidx_v.at[u % 4]], rows_v.at[u % 3], semr[u % 3])

    fire_idx(0)
    fire_idx(1)
    fire_idx(2)
    fire_rows(0)
    fire_rows(1)
    for u in range(UNITS):
        if u + 3 < UNITS:
            fire_idx(u + 3)
        d_r[u].wait()
        if u + 2 < UNITS:
            fire_rows(u + 2)
        sidx_u = sidx_v.at[u % 4]
        didx_u = didx_v.at[u % 4]
        rows_u = rows_v.at[u % 3]

        @plsc.parallel_loop(0, UG, unroll=1)
        def group_body(g):
            r0 = g * 16
            row_ids = r0 + lane
            si = sidx_u[pl.ds(r0, 16)]
            di = didx_u[pl.ds(r0, 16)]
            s = plsc.load_gather(s_loc, [si])
            d = plsc.load_gather(d_loc, [di])
            t = s + d
            ex = jnp.exp(jnp.maximum(t, 0.2 * t))
            plsc.addupdate_scatter(
                acc_v, [jnp.full((16,), D_HID, _i32), di], ex)
            for c in range(D_HID):
                hc = plsc.load_gather(
                    rows_u, [row_ids, jnp.full((16,), c, _i32)])
                plsc.addupdate_scatter(
                    acc_v, [jnp.full((16,), c, _i32), di], ex * hc)

    pltpu.sync_copy(acc_v, out_hbm.at[w])


def _stage3_body(part_ref, nt1_ref, b1_ref, wed_ref, w2_ref, a2s_ref,
                 a2d_ref, rep_ref, nt2_ref, acc_ref):
    i = pl.program_id(0)

    @pl.when(i == 0)
    def _():
        acc_ref[...] = jnp.zeros((W_ACC, N), _f32)

    acc_ref[...] += jnp.sum(part_ref[...], axis=0)

    @pl.when(i == (NW // NP) - 1)
    def _():
        red = acc_ref[...]
        s1 = nt1_ref[0:1, :]
        d1 = nt1_ref[1:2, :]
        h1 = nt1_ref[2:10, :]
        t = s1 + d1
        ex = jnp.exp(jnp.maximum(t, 0.2 * t))
        num = red[0:D_HID] + ex * h1
        den = red[D_HID:W_ACC] + ex
        enc = num / (den + 1e-16) + b1_ref[...]
        enc = _leaky(enc, 0.01)
        rep = lax.dot_general(wed_ref[...], enc, (((0,), (0,)), ((), ())),
                              preferred_element_type=_f32)
        c_s = jnp.dot(w2_ref[...], a2s_ref[...],
                      preferred_element_type=_f32)
        c_d = jnp.dot(w2_ref[...], a2d_ref[...],
                      preferred_element_type=_f32)
        s2 = jnp.sum(rep * c_s, axis=0, keepdims=True)
        d2 = jnp.sum(rep * c_d, axis=0, keepdims=True)
        rep_ref[...] = rep.T
        nt2_ref[...] = jnp.concatenate([s2, d2, rep], axis=0)


def _stage3(part, nt1, b1, wed, w2, a2s, a2d):
    return pl.pallas_call(
        _stage3_body,
        grid=(NW // NP,),
        in_specs=[
            pl.BlockSpec((NP, W_ACC, N), lambda i: (i, 0, 0)),
            pl.BlockSpec((10, N), lambda i: (0, 0)),
            pl.BlockSpec((D_HID, 1), lambda i: (0, 0)),
            pl.BlockSpec((D_HID, D_HID), lambda i: (0, 0)),
            pl.BlockSpec((D_HID, D_IN), lambda i: (0, 0)),
            pl.BlockSpec((D_IN, 1), lambda i: (0, 0)),
            pl.BlockSpec((D_IN, 1), lambda i: (0, 0)),
        ],
        out_specs=[
            pl.BlockSpec((N, D_HID), lambda i: (0, 0)),
            pl.BlockSpec((10, N), lambda i: (0, 0)),
        ],
        out_shape=[
            jax.ShapeDtypeStruct((N, D_HID), _f32),
            jax.ShapeDtypeStruct((10, N), _f32),
        ],
        scratch_shapes=[pltpu.VMEM((W_ACC, N), _f32)],
    )(part, nt1, b1.reshape(D_HID, 1), wed, w2,
      a2s.reshape(D_IN, 1), a2d.reshape(D_IN, 1))


def _stage5_body(part_ref, nt2_ref, w2_ref, b2_ref, out_ref, acc_ref):
    i = pl.program_id(0)

    @pl.when(i == 0)
    def _():
        acc_ref[...] = jnp.zeros((W_ACC, N), _f32)

    acc_ref[...] += jnp.sum(part_ref[...], axis=0)

    @pl.when(i == (NW // NP) - 1)
    def _():
        red = acc_ref[...]
        s2 = nt2_ref[0:1, :]
        d2 = nt2_ref[1:2, :]
        rep = nt2_ref[2:10, :]
        t = s2 + d2
        ex = jnp.exp(jnp.maximum(t, 0.2 * t))
        agg_t = (red[0:D_HID] + ex * rep) / (red[D_HID:W_ACC] + ex + 1e-16)
        agg = agg_t.T
        y = jnp.dot(agg, w2_ref[...], preferred_element_type=_f32)
        y = y + b2_ref[...]
        out_ref[...] = _leaky(y, 0.01)


def _stage5(part, nt2, w2, b2):
    return pl.pallas_call(
        _stage5_body,
        grid=(NW // NP,),
        in_specs=[
            pl.BlockSpec((NP, W_ACC, N), lambda i: (i, 0, 0)),
            pl.BlockSpec((10, N), lambda i: (0, 0)),
            pl.BlockSpec((D_HID, D_IN), lambda i: (0, 0)),
            pl.BlockSpec((1, D_IN), lambda i: (0, 0)),
        ],
        out_specs=pl.BlockSpec((N, D_IN), lambda i: (0, 0)),
        out_shape=jax.ShapeDtypeStruct((N, D_IN), _f32),
        scratch_shapes=[pltpu.VMEM((W_ACC, N), _f32)],
    )(part, nt2, w2, b2.reshape(1, D_IN))


def kernel(x, edge_index, W1, a1_src, a1_dst, b1, W_ed, W2, a2_src, a2_dst,
           b2):
    edges = edge_index.astype(_i32)
    h1, nt1 = _stage1(x, W1, a1_src, a1_dst)
    part1 = _edge_pass(h1, nt1, edges)
    rep, nt2 = _stage3(part1, nt1, b1, W_ed, W2, a2_src, a2_dst)
    part2 = _edge_pass(rep, nt2, edges)
    return _stage5(part2, nt2, W2, b2)


# speedup vs baseline: 118.7576x; 1.0002x over previous
import functools

import jax
import jax.numpy as jnp
from jax import lax
from jax.experimental import pallas as pl
from jax.experimental.pallas import tpu as pltpu
from jax.experimental.pallas import tpu_sc as plsc

N = 10000
E = 320000
D_IN = 128
D_HID = 8
NC = 2
NS = 16
NW = NC * NS
EPW = E // NW
UB = 400
UNITS = EPW // UB
UG = UB // 16
W_ACC = D_HID + 1
NP = 8
_f32 = jnp.float32
_i32 = jnp.int32


def _leaky(v, slope):
    return jnp.maximum(v, slope * v)


def _stage1_body(x_ref, w1_ref, as_ref, ad_ref, h_ref, nt_ref):
    h = jnp.dot(x_ref[...], w1_ref[...], preferred_element_type=_f32)
    s = jnp.dot(h, as_ref[...], preferred_element_type=_f32)
    d = jnp.dot(h, ad_ref[...], preferred_element_type=_f32)
    h_ref[...] = h
    nt_ref[...] = jnp.concatenate([s, d, h], axis=1).T


def _stage1(x, w1, a_s, a_d):
    return pl.pallas_call(
        _stage1_body,
        out_shape=[
            jax.ShapeDtypeStruct((N, D_HID), _f32),
            jax.ShapeDtypeStruct((10, N), _f32),
        ],
    )(x, w1, a_s.reshape(D_HID, 1), a_d.reshape(D_HID, 1))


_mesh = plsc.VectorSubcoreMesh(
    core_axis_name="c", subcore_axis_name="s", num_cores=NC, num_subcores=NS)


@functools.partial(
    pl.kernel,
    out_type=jax.ShapeDtypeStruct((NW, W_ACC, N), _f32),
    mesh=_mesh,
    compiler_params=pltpu.CompilerParams(
        needs_layout_passes=False, use_tc_tiling_on_sc=False),
    scratch_types=[
        pltpu.VMEM((W_ACC, N), _f32),
        pltpu.VMEM((N,), _f32),
        pltpu.VMEM((N,), _f32),
        pltpu.VMEM((5, UB), _i32),
        pltpu.VMEM((5, UB), _i32),
        pltpu.VMEM((4, UB, D_HID), _f32),
        pltpu.SemaphoreType.DMA,
        pltpu.SemaphoreType.DMA,
        pltpu.SemaphoreType.DMA,
        pltpu.SemaphoreType.DMA,
        pltpu.SemaphoreType.DMA,
        pltpu.SemaphoreType.DMA,
        pltpu.SemaphoreType.DMA,
        pltpu.SemaphoreType.DMA,
        pltpu.SemaphoreType.DMA,
        pltpu.SemaphoreType.DMA,
    ],
)
def _edge_pass(h_hbm, nt_hbm, edge_hbm, out_hbm,
               acc_v, s_loc, d_loc, sidx_v, didx_v, rows_v,
               semi0, semi1, semi2, semi3, semi4,
               semr0, semr1, semr2, semr3, semt):
    cid = lax.axis_index("c")
    sid = lax.axis_index("s")
    w = sid * NC + cid
    semi = [semi0, semi1, semi2, semi3, semi4]
    semr = [semr0, semr1, semr2, semr3]

    d_s = pltpu.async_copy(nt_hbm.at[0], s_loc, semt)
    d_d = pltpu.async_copy(nt_hbm.at[1], d_loc, semt)

    @plsc.parallel_loop(0, N // 16, unroll=4)
    def zero_body(i):
        for c in range(W_ACC):
            acc_v[c, pl.ds(i * 16, 16)] = jnp.zeros((16,), _f32)

    d_s.wait()
    d_d.wait()

    lane = jnp.arange(16, dtype=_i32)
    ebase = w * EPW
    d_is = [None] * UNITS
    d_id = [None] * UNITS
    d_r = [None] * UNITS

    HB = UB // 2

    def fire_idx(u):
        j = u % 5
        base = pl.multiple_of(ebase + u * UB, UB)
        d_is[u] = pltpu.async_copy(
            edge_hbm.at[0, pl.ds(base, UB)], sidx_v.at[j], semi[j])
        d_id[u] = pltpu.async_copy(
            edge_hbm.at[1, pl.ds(base, UB)], didx_v.at[j], semi[j])

    def fire_rows(u):
        d_is[u].wait()
        d_id[u].wait()
        j = u % 5
        jr = u % 4
        d_r[u] = [
            pltpu.async_copy(
                h_hbm.at[sidx_v.at[j, pl.ds(0, HB)]],
                rows_v.at[jr, pl.ds(0, HB)], semr[jr]),
            pltpu.async_copy(
                h_hbm.at[sidx_v.at[j, pl.ds(HB, HB)]],
                rows_v.at[jr, pl.ds(HB, HB)], semr[jr]),
        ]

    fire_idx(0)
    fire_idx(1)
    fire_idx(2)
    fire_idx(3)
    fire_rows(0)
    fire_rows(1)
    fire_rows(2)
    for u in range(UNITS):
        if u + 4 < UNITS:
            fire_idx(u + 4)
        d_r[u][0].wait()
        d_r[u][1].wait()
        if u + 3 < UNITS:
            fire_rows(u + 3)
        sidx_u = sidx_v.at[u % 5]
        didx_u = didx_v.at[u % 5]
        rows_u = rows_v.at[u % 4]

        @plsc.parallel_loop(0, UG, unroll=1)
        def group_body(g):
            r0 = g * 16
            row_ids = r0 + lane
            si = sidx_u[pl.ds(r0, 16)]
            di = didx_u[pl.ds(r0, 16)]
            s = plsc.load_gather(s_loc, [si])
            d = plsc.load_gather(d_loc, [di])
            t = s + d
            ex = jnp.exp(jnp.maximum(t, 0.2 * t))
            plsc.addupdate_scatter(
                acc_v, [jnp.full((16,), D_HID, _i32), di], ex)
            for c in range(D_HID):
                hc = plsc.load_gather(
                    rows_u, [row_ids, jnp.full((16,), c, _i32)])
                plsc.addupdate_scatter(
                    acc_v, [jnp.full((16,), c, _i32), di], ex * hc)

    pltpu.sync_copy(acc_v, out_hbm.at[w])


def _stage3_body(part_ref, nt1_ref, b1_ref, wed_ref, w2_ref, a2s_ref,
                 a2d_ref, rep_ref, nt2_ref, acc_ref):
    i = pl.program_id(0)

    @pl.when(i == 0)
    def _():
        acc_ref[...] = jnp.zeros((W_ACC, N), _f32)

    acc_ref[...] += jnp.sum(part_ref[...], axis=0)

    @pl.when(i == (NW // NP) - 1)
    def _():
        red = acc_ref[...]
        s1 = nt1_ref[0:1, :]
        d1 = nt1_ref[1:2, :]
        h1 = nt1_ref[2:10, :]
        t = s1 + d1
        ex = jnp.exp(jnp.maximum(t, 0.2 * t))
        num = red[0:D_HID] + ex * h1
        den = red[D_HID:W_ACC] + ex
        enc = num / (den + 1e-16) + b1_ref[...]
        enc = _leaky(enc, 0.01)
        rep = lax.dot_general(wed_ref[...], enc, (((0,), (0,)), ((), ())),
                              preferred_element_type=_f32)
        c_s = jnp.dot(w2_ref[...], a2s_ref[...],
                      preferred_element_type=_f32)
        c_d = jnp.dot(w2_ref[...], a2d_ref[...],
                      preferred_element_type=_f32)
        s2 = jnp.sum(rep * c_s, axis=0, keepdims=True)
        d2 = jnp.sum(rep * c_d, axis=0, keepdims=True)
        rep_ref[...] = rep.T
        nt2_ref[...] = jnp.concatenate([s2, d2, rep], axis=0)


def _stage3(part, nt1, b1, wed, w2, a2s, a2d):
    return pl.pallas_call(
        _stage3_body,
        grid=(NW // NP,),
        in_specs=[
            pl.BlockSpec((NP, W_ACC, N), lambda i: (i, 0, 0)),
            pl.BlockSpec((10, N), lambda i: (0, 0)),
            pl.BlockSpec((D_HID, 1), lambda i: (0, 0)),
            pl.BlockSpec((D_HID, D_HID), lambda i: (0, 0)),
            pl.BlockSpec((D_HID, D_IN), lambda i: (0, 0)),
            pl.BlockSpec((D_IN, 1), lambda i: (0, 0)),
            pl.BlockSpec((D_IN, 1), lambda i: (0, 0)),
        ],
        out_specs=[
            pl.BlockSpec((N, D_HID), lambda i: (0, 0)),
            pl.BlockSpec((10, N), lambda i: (0, 0)),
        ],
        out_shape=[
            jax.ShapeDtypeStruct((N, D_HID), _f32),
            jax.ShapeDtypeStruct((10, N), _f32),
        ],
        scratch_shapes=[pltpu.VMEM((W_ACC, N), _f32)],
    )(part, nt1, b1.reshape(D_HID, 1), wed, w2,
      a2s.reshape(D_IN, 1), a2d.reshape(D_IN, 1))


def _stage5_body(part_ref, nt2_ref, w2_ref, b2_ref, out_ref, acc_ref):
    i = pl.program_id(0)

    @pl.when(i == 0)
    def _():
        acc_ref[...] = jnp.zeros((W_ACC, N), _f32)

    acc_ref[...] += jnp.sum(part_ref[...], axis=0)

    @pl.when(i == (NW // NP) - 1)
    def _():
        red = acc_ref[...]
        s2 = nt2_ref[0:1, :]
        d2 = nt2_ref[1:2, :]
        rep = nt2_ref[2:10, :]
        t = s2 + d2
        ex = jnp.exp(jnp.maximum(t, 0.2 * t))
        agg_t = (red[0:D_HID] + ex * rep) / (red[D_HID:W_ACC] + ex + 1e-16)
        agg = agg_t.T
        y = jnp.dot(agg, w2_ref[...], preferred_element_type=_f32)
        y = y + b2_ref[...]
        out_ref[...] = _leaky(y, 0.01)


def _stage5(part, nt2, w2, b2):
    return pl.pallas_call(
        _stage5_body,
        grid=(NW // NP,),
        in_specs=[
            pl.BlockSpec((NP, W_ACC, N), lambda i: (i, 0, 0)),
            pl.BlockSpec((10, N), lambda i: (0, 0)),
            pl.BlockSpec((D_HID, D_IN), lambda i: (0, 0)),
            pl.BlockSpec((1, D_IN), lambda i: (0, 0)),
        ],
        out_specs=pl.BlockSpec((N, D_IN), lambda i: (0, 0)),
        out_shape=jax.ShapeDtypeStruct((N, D_IN), _f32),
        scratch_shapes=[pltpu.VMEM((W_ACC, N), _f32)],
    )(part, nt2, w2, b2.reshape(1, D_IN))


def kernel(x, edge_index, W1, a1_src, a1_dst, b1, W_ed, W2, a2_src, a2_dst,
           b2):
    edges = edge_index.astype(_i32)
    h1, nt1 = _stage1(x, W1, a1_src, a1_dst)
    part1 = _edge_pass(h1, nt1, edges)
    rep, nt2 = _stage3(part1, nt1, b1, W_ed, W2, a2_src, a2_dst)
    part2 = _edge_pass(rep, nt2, edges)
    return _stage5(part2, nt2, W2, b2)
